# baseline (speedup 1.0000x reference)
import functools

import jax
import jax.numpy as jnp
from jax import lax
from jax.experimental import pallas as pl
from jax.experimental.pallas import tpu as pltpu
from jax.experimental.pallas import tpu_sc as plsc


def _sc_gather(ids, adj, emb):
    B = ids.shape[0]
    DEG = adj.shape[1]
    D = emb.shape[1]
    info = plsc.get_sparse_core_info()
    NW = info.num_cores * info.num_subcores
    IPW = B // NW
    C1 = IPW * DEG
    LCHUNK = 128
    NCH = C1 // LCHUNK
    C2 = C1 * DEG
    NCH2 = C2 // LCHUNK
    NPAIR = NCH2 // 2
    GPC = LCHUNK // DEG
    mesh = plsc.VectorSubcoreMesh(core_axis_name="c", subcore_axis_name="s")

    @functools.partial(
        pl.kernel,
        mesh=mesh,
        compiler_params=pltpu.CompilerParams(use_tc_tiling_on_sc=False),
        out_type=(
            jax.ShapeDtypeStruct((B * DEG,), jnp.int32),
            jax.ShapeDtypeStruct((B * DEG * DEG,), jnp.int32),
        ),
        scratch_types=[
            pltpu.VMEM((IPW,), jnp.int32),
            pltpu.VMEM((IPW, DEG), jnp.int32),
            pltpu.VMEM((C1,), jnp.int32),
            pltpu.VMEM((C1, DEG), jnp.int32),
            pltpu.VMEM((C2,), jnp.int32),
            pltpu.SemaphoreType.DMA,
        ],
    )
    def index_body(ids_hbm, adj_hbm, idx1_out, idx2_out,
                   ids_v, n1_v, idx1_v, n2buf, idx2_v, semH):
        wid = lax.axis_index("c") * info.num_subcores + lax.axis_index("s")
        base = wid * IPW
        pltpu.sync_copy(ids_hbm.at[pl.ds(base, IPW)], ids_v)
        pltpu.async_copy(adj_hbm.at[ids_v], n1_v, semH).wait()
        for i in range(IPW):
            idx1_v[pl.ds(i * DEG, DEG)] = n1_v[i, :]
        for c in range(NCH):
            pltpu.async_copy(adj_hbm.at[idx1_v.at[pl.ds(c * LCHUNK, LCHUNK)]],
                             n2buf.at[pl.ds(c * LCHUNK, LCHUNK)], semH)
        for c in range(NCH):
            pltpu.make_async_copy(
                adj_hbm.at[idx1_v.at[pl.ds(0, LCHUNK)]],
                n2buf.at[pl.ds(c * LCHUNK, LCHUNK)], semH).wait()
        pltpu.sync_copy(idx1_v, idx1_out.at[pl.ds(wid * C1, C1)])

        def fl_body(j, carry):
            for jj in range(4):
                r = j * 4 + jj
                idx2_v[pl.ds(r * DEG, DEG)] = n2buf[r, :]
            return carry
        lax.fori_loop(0, C1 // 4, fl_body, 0)
        pltpu.sync_copy(idx2_v, idx2_out.at[pl.ds(wid * C2, C2)])

    @functools.partial(
        pl.kernel,
        mesh=mesh,
        compiler_params=pltpu.CompilerParams(use_tc_tiling_on_sc=False),
        out_type=(
            jax.ShapeDtypeStruct((B * DEG, D), jnp.float32),
            jax.ShapeDtypeStruct((B * DEG, D), jnp.float32),
        ),
        scratch_types=[
            pltpu.VMEM((C1,), jnp.int32),
            pltpu.VMEM((C2,), jnp.int32),
            pltpu.VMEM((C1, D), jnp.float32),
            pltpu.VMEM((C1, D), jnp.float32),
            pltpu.VMEM((LCHUNK, D), jnp.float32),
            pltpu.VMEM((LCHUNK, D), jnp.float32),
            pltpu.VMEM((LCHUNK, D), jnp.float32),
            pltpu.VMEM((LCHUNK, D), jnp.float32),
            pltpu.SemaphoreType.DMA,
            pltpu.SemaphoreType.DMA,
            pltpu.SemaphoreType.DMA,
            pltpu.SemaphoreType.DMA,
            pltpu.SemaphoreType.DMA,
            pltpu.SemaphoreType.DMA,
        ],
    )
    def fetch_body(idx1_hbm, idx2_hbm, emb_hbm, f1_out, m2_out,
                   idx1_v, idx2_v, f1buf, m2buf,
                   gbuf0, gbuf1, gbuf2, gbuf3,
                   semH, sem0, sem1, sem2, sem3, semO):
        wid = lax.axis_index("c") * info.num_subcores + lax.axis_index("s")
        pltpu.sync_copy(idx1_hbm.at[pl.ds(wid * C1, C1)], idx1_v)
        pltpu.sync_copy(idx2_hbm.at[pl.ds(wid * C2, C2)], idx2_v)
        for c in range(NCH):
            pltpu.async_copy(emb_hbm.at[idx1_v.at[pl.ds(c * LCHUNK, LCHUNK)]],
                             f1buf.at[pl.ds(c * LCHUNK, LCHUNK)], semH)
        for c in range(NCH):
            pltpu.make_async_copy(
                emb_hbm.at[idx1_v.at[pl.ds(0, LCHUNK)]],
                f1buf.at[pl.ds(c * LCHUNK, LCHUNK)], semH).wait()
        f1_wb = pltpu.make_async_copy(f1buf, f1_out.at[pl.ds(wid * C1, C1)],
                                      semO)
        f1_wb.start()

        def start2(c, buf, sem):
            pltpu.async_copy(
                emb_hbm.at[idx2_v.at[pl.ds(c * LCHUNK, LCHUNK)]], buf, sem)

        def wait2(buf, sem):
            pltpu.make_async_copy(
                emb_hbm.at[idx2_v.at[pl.ds(0, LCHUNK)]], buf, sem).wait()

        def rne(v):
            u = lax.bitcast_convert_type(v, jnp.uint32)
            u = (u + jnp.uint32(0x8000)) & jnp.uint32(0xFFFF0000)
            return lax.bitcast_convert_type(u, jnp.float32)

        def accum(buf, row_base):
            def g_body(g, carry):
                for d in range(D // 16):
                    sl = pl.ds(d * 16, 16)
                    vals = [rne(buf[g * DEG + r, sl]) for r in range(DEG)]
                    while len(vals) > 1:
                        vals = [vals[i] + vals[i + 1]
                                for i in range(0, len(vals), 2)]
                    m2buf[row_base + g, sl] = vals[0]
                return carry
            lax.fori_loop(0, GPC, g_body, 0)

        bufs = (gbuf0, gbuf1, gbuf2, gbuf3)
        sems = (sem0, sem1, sem2, sem3)
        for b in range(3):
            start2(b, bufs[b], sems[b])

        def t_body(t, carry):
            for b in range(4):
                c = 4 * t + b
                start2(c + 3, bufs[(b + 3) % 4], sems[(b + 3) % 4])
                wait2(bufs[b], sems[b])
                accum(bufs[b], c * GPC)
            return carry

        lax.fori_loop(0, NCH2 // 4 - 1, t_body, 0)
        for b in range(4):
            c = NCH2 - 4 + b
            if c + 3 < NCH2:
                start2(c + 3, bufs[(b + 3) % 4], sems[(b + 3) % 4])
            wait2(bufs[b], sems[b])
            accum(bufs[b], c * GPC)
        pltpu.sync_copy(m2buf, m2_out.at[pl.ds(wid * C1, C1)])
        f1_wb.wait()

    idx1, idx2 = index_body(ids, adj)
    return fetch_body(idx1, idx2, emb)


def _tc_dense(f1raw, m2sum, x0row, prep_W, prep_b, a1_Wx, a1_Wn, a2_Wx, a2_Wn,
              fc_W, fc_b, deg):
    N, D = f1raw.shape
    R = 4096
    NB = N // R
    G = R // deg
    B = N // deg
    H = a1_Wx.shape[1]

    def body(f1_ref, m2_ref, x0_ref, pW, pb, wx, wn, w2x, w2n, fw, fb,
             out_ref, hn_acc, m1_acc):
        i = pl.program_id(0)
        hidot = functools.partial(jnp.dot, precision=lax.Precision.HIGHEST)
        pWr = pW[...].astype(jnp.bfloat16).astype(jnp.float32)
        f1 = jnp.dot(f1_ref[...], pW[...]) + pb[...]
        m2 = hidot(m2_ref[...] * (1.0 / deg), pWr) + pb[...]
        h1 = jnp.concatenate([jnp.dot(f1, wx[...]), jnp.dot(m2, wn[...])],
                             axis=1)
        h1 = jnp.maximum(h1, 0.0)
        h1m = jnp.mean(h1.reshape(G, deg, 2 * H), axis=1)
        hn_acc[pl.ds(i * G, G), :] = jnp.dot(h1m, w2n[...])
        m1_acc[pl.ds(i * G, G), :] = jnp.mean(f1.reshape(G, deg, D), axis=1)

        @pl.when(i == NB - 1)
        def _():
            m1 = m1_acc[...]
            x0 = jnp.dot(x0_ref[...], pW[...]) + pb[...]
            xl = jnp.broadcast_to(jnp.dot(x0, wx[...]), (B, H))
            h0 = jnp.concatenate([xl, jnp.dot(m1, wn[...])], axis=1)
            h0 = jnp.maximum(h0, 0.0)
            g = jnp.concatenate([jnp.dot(h0, w2x[...]), hn_acc[...]], axis=1)
            nrm = jnp.maximum(
                jnp.sqrt(jnp.sum(g * g, axis=1, keepdims=True)), 1e-12)
            out_ref[...] = jnp.dot(g / nrm, fw[...]) + fb[...]

    return pl.pallas_call(
        body,
        grid=(NB,),
        in_specs=[
            pl.BlockSpec((R, D), lambda i: (i, 0)),
            pl.BlockSpec((R, D), lambda i: (i, 0)),
            pl.BlockSpec((1, D), lambda i: (0, 0)),
            pl.BlockSpec((D, D), lambda i: (0, 0)),
            pl.BlockSpec((1, D), lambda i: (0, 0)),
            pl.BlockSpec((D, H), lambda i: (0, 0)),
            pl.BlockSpec((D, H), lambda i: (0, 0)),
            pl.BlockSpec((2 * H, H), lambda i: (0, 0)),
            pl.BlockSpec((2 * H, H), lambda i: (0, 0)),
            pl.BlockSpec((2 * H, 1), lambda i: (0, 0)),
            pl.BlockSpec((1, 1), lambda i: (0, 0)),
        ],
        out_specs=pl.BlockSpec((B, 1), lambda i: (0, 0)),
        out_shape=jax.ShapeDtypeStruct((B, 1), jnp.float32),
        scratch_shapes=[
            pltpu.VMEM((B, H), jnp.float32),
            pltpu.VMEM((B, D), jnp.float32),
        ],
    )(f1raw, m2sum, x0row, prep_W, prep_b.reshape(1, D), a1_Wx, a1_Wn,
      a2_Wx, a2_Wn, fc_W, fc_b.reshape(1, 1))


def kernel(ids, adj, emb, prep_W, prep_b, a1_Wx, a1_Wn, a2_Wx, a2_Wn, fc_W, fc_b):
    ids = ids.astype(jnp.int32)
    adj = adj.astype(jnp.int32)
    f1raw, m2sum = _sc_gather(ids, adj, emb)
    x0row = lax.slice(emb, (emb.shape[0] - 1, 0), (emb.shape[0], emb.shape[1]))
    return _tc_dense(f1raw, m2sum, x0row, prep_W, prep_b, a1_Wx, a1_Wn,
                     a2_Wx, a2_Wn, fc_W, fc_b, adj.shape[1])


# speedup vs baseline: 1.0143x; 1.0143x over previous
import functools

import jax
import jax.numpy as jnp
from jax import lax
from jax.experimental import pallas as pl
from jax.experimental.pallas import tpu as pltpu
from jax.experimental.pallas import tpu_sc as plsc


def _sc_gather(ids, adj, emb):
    B = ids.shape[0]
    DEG = adj.shape[1]
    D = emb.shape[1]
    info = plsc.get_sparse_core_info()
    NW = info.num_cores * info.num_subcores
    IPW = B // NW
    C1 = IPW * DEG
    LCHUNK = 128
    NCH = C1 // LCHUNK
    C2 = C1 * DEG
    NCH2 = C2 // LCHUNK
    NPAIR = NCH2 // 2
    GPC = LCHUNK // DEG
    mesh = plsc.VectorSubcoreMesh(core_axis_name="c", subcore_axis_name="s")

    @functools.partial(
        pl.kernel,
        mesh=mesh,
        compiler_params=pltpu.CompilerParams(use_tc_tiling_on_sc=False),
        out_type=(
            jax.ShapeDtypeStruct((B * DEG,), jnp.int32),
            jax.ShapeDtypeStruct((B * DEG * DEG,), jnp.int32),
        ),
        scratch_types=[
            pltpu.VMEM((IPW,), jnp.int32),
            pltpu.VMEM((IPW, DEG), jnp.int32),
            pltpu.VMEM((C1,), jnp.int32),
            pltpu.VMEM((C1, DEG), jnp.int32),
            pltpu.VMEM((C2,), jnp.int32),
            pltpu.SemaphoreType.DMA,
        ],
    )
    def index_body(ids_hbm, adj_hbm, idx1_out, idx2_out,
                   ids_v, n1_v, idx1_v, n2buf, idx2_v, semH):
        wid = lax.axis_index("c") * info.num_subcores + lax.axis_index("s")
        base = wid * IPW
        pltpu.sync_copy(ids_hbm.at[pl.ds(base, IPW)], ids_v)
        pltpu.async_copy(adj_hbm.at[ids_v], n1_v, semH).wait()
        for i in range(IPW):
            idx1_v[pl.ds(i * DEG, DEG)] = n1_v[i, :]
        for c in range(NCH):
            pltpu.async_copy(adj_hbm.at[idx1_v.at[pl.ds(c * LCHUNK, LCHUNK)]],
                             n2buf.at[pl.ds(c * LCHUNK, LCHUNK)], semH)
        for c in range(NCH):
            pltpu.make_async_copy(
                adj_hbm.at[idx1_v.at[pl.ds(0, LCHUNK)]],
                n2buf.at[pl.ds(c * LCHUNK, LCHUNK)], semH).wait()
        pltpu.sync_copy(idx1_v, idx1_out.at[pl.ds(wid * C1, C1)])

        def fl_body(j, carry):
            for jj in range(4):
                r = j * 4 + jj
                idx2_v[pl.ds(r * DEG, DEG)] = n2buf[r, :]
            return carry
        lax.fori_loop(0, C1 // 4, fl_body, 0)
        pltpu.sync_copy(idx2_v, idx2_out.at[pl.ds(wid * C2, C2)])

    @functools.partial(
        pl.kernel,
        mesh=mesh,
        compiler_params=pltpu.CompilerParams(use_tc_tiling_on_sc=False),
        out_type=(
            jax.ShapeDtypeStruct((B * DEG, D), jnp.float32),
            jax.ShapeDtypeStruct((B * DEG, D), jnp.float32),
        ),
        scratch_types=[
            pltpu.VMEM((C1,), jnp.int32),
            pltpu.VMEM((C2,), jnp.int32),
            pltpu.VMEM((C1, D), jnp.float32),
            pltpu.VMEM((C1, D), jnp.float32),
            pltpu.VMEM((LCHUNK, D), jnp.float32),
            pltpu.VMEM((LCHUNK, D), jnp.float32),
            pltpu.VMEM((LCHUNK, D), jnp.float32),
            pltpu.VMEM((LCHUNK, D), jnp.float32),
            pltpu.SemaphoreType.DMA,
            pltpu.SemaphoreType.DMA,
            pltpu.SemaphoreType.DMA,
            pltpu.SemaphoreType.DMA,
            pltpu.SemaphoreType.DMA,
            pltpu.SemaphoreType.DMA,
        ],
    )
    def fetch_body(idx1_hbm, idx2_hbm, emb_hbm, f1_out, m2_out,
                   idx1_v, idx2_v, f1buf, m2buf,
                   gbuf0, gbuf1, gbuf2, gbuf3,
                   semH, sem0, sem1, sem2, sem3, semO):
        wid = lax.axis_index("c") * info.num_subcores + lax.axis_index("s")
        pltpu.sync_copy(idx2_hbm.at[pl.ds(wid * C2, C2)], idx2_v)
        pltpu.sync_copy(idx1_hbm.at[pl.ds(wid * C1, C1)], idx1_v)

        def start2(c, buf, sem):
            pltpu.async_copy(
                emb_hbm.at[idx2_v.at[pl.ds(c * LCHUNK, LCHUNK)]], buf, sem)

        def wait2(buf, sem):
            pltpu.make_async_copy(
                emb_hbm.at[idx2_v.at[pl.ds(0, LCHUNK)]], buf, sem).wait()

        def rne(v):
            u = lax.bitcast_convert_type(v, jnp.uint32)
            u = (u + jnp.uint32(0x8000)) & jnp.uint32(0xFFFF0000)
            return lax.bitcast_convert_type(u, jnp.float32)

        def accum(buf, row_base):
            def g_body(g, carry):
                for d in range(D // 16):
                    sl = pl.ds(d * 16, 16)
                    vals = [rne(buf[g * DEG + r, sl]) for r in range(DEG)]
                    while len(vals) > 1:
                        vals = [vals[i] + vals[i + 1]
                                for i in range(0, len(vals), 2)]
                    m2buf[row_base + g, sl] = vals[0]
                return carry
            lax.fori_loop(0, GPC, g_body, 0)

        bufs = (gbuf0, gbuf1, gbuf2, gbuf3)
        sems = (sem0, sem1, sem2, sem3)
        for b in range(3):
            start2(b, bufs[b], sems[b])
        for c in range(NCH):
            pltpu.async_copy(emb_hbm.at[idx1_v.at[pl.ds(c * LCHUNK, LCHUNK)]],
                             f1buf.at[pl.ds(c * LCHUNK, LCHUNK)], semH)

        def t_body(t, carry):
            for b in range(4):
                c = 4 * t + b
                start2(c + 3, bufs[(b + 3) % 4], sems[(b + 3) % 4])
                wait2(bufs[b], sems[b])
                accum(bufs[b], c * GPC)
            return carry

        lax.fori_loop(0, NCH2 // 4 - 1, t_body, 0)
        for b in range(4):
            c = NCH2 - 4 + b
            if c + 3 < NCH2:
                start2(c + 3, bufs[(b + 3) % 4], sems[(b + 3) % 4])
            wait2(bufs[b], sems[b])
            accum(bufs[b], c * GPC)
        for c in range(NCH):
            pltpu.make_async_copy(
                emb_hbm.at[idx1_v.at[pl.ds(0, LCHUNK)]],
                f1buf.at[pl.ds(c * LCHUNK, LCHUNK)], semH).wait()
        f1_wb = pltpu.make_async_copy(f1buf, f1_out.at[pl.ds(wid * C1, C1)],
                                      semO)
        f1_wb.start()
        pltpu.sync_copy(m2buf, m2_out.at[pl.ds(wid * C1, C1)])
        f1_wb.wait()

    idx1, idx2 = index_body(ids, adj)
    return fetch_body(idx1, idx2, emb)


def _tc_dense(f1raw, m2sum, x0row, prep_W, prep_b, a1_Wx, a1_Wn, a2_Wx, a2_Wn,
              fc_W, fc_b, deg):
    N, D = f1raw.shape
    R = 4096
    NB = N // R
    G = R // deg
    B = N // deg
    H = a1_Wx.shape[1]

    def body(f1_ref, m2_ref, x0_ref, pW, pb, wx, wn, w2x, w2n, fw, fb,
             out_ref, hn_acc, m1_acc):
        i = pl.program_id(0)
        hidot = functools.partial(jnp.dot, precision=lax.Precision.HIGHEST)
        pWr = pW[...].astype(jnp.bfloat16).astype(jnp.float32)
        f1 = jnp.dot(f1_ref[...], pW[...]) + pb[...]
        m2 = hidot(m2_ref[...] * (1.0 / deg), pWr) + pb[...]
        h1 = jnp.concatenate([jnp.dot(f1, wx[...]), jnp.dot(m2, wn[...])],
                             axis=1)
        h1 = jnp.maximum(h1, 0.0)
        h1m = jnp.mean(h1.reshape(G, deg, 2 * H), axis=1)
        hn_acc[pl.ds(i * G, G), :] = jnp.dot(h1m, w2n[...])
        m1_acc[pl.ds(i * G, G), :] = jnp.mean(f1.reshape(G, deg, D), axis=1)

        @pl.when(i == NB - 1)
        def _():
            m1 = m1_acc[...]
            x0 = jnp.dot(x0_ref[...], pW[...]) + pb[...]
            xl = jnp.broadcast_to(jnp.dot(x0, wx[...]), (B, H))
            h0 = jnp.concatenate([xl, jnp.dot(m1, wn[...])], axis=1)
            h0 = jnp.maximum(h0, 0.0)
            g = jnp.concatenate([jnp.dot(h0, w2x[...]), hn_acc[...]], axis=1)
            nrm = jnp.maximum(
                jnp.sqrt(jnp.sum(g * g, axis=1, keepdims=True)), 1e-12)
            out_ref[...] = jnp.dot(g / nrm, fw[...]) + fb[...]

    return pl.pallas_call(
        body,
        grid=(NB,),
        in_specs=[
            pl.BlockSpec((R, D), lambda i: (i, 0)),
            pl.BlockSpec((R, D), lambda i: (i, 0)),
            pl.BlockSpec((1, D), lambda i: (0, 0)),
            pl.BlockSpec((D, D), lambda i: (0, 0)),
            pl.BlockSpec((1, D), lambda i: (0, 0)),
            pl.BlockSpec((D, H), lambda i: (0, 0)),
            pl.BlockSpec((D, H), lambda i: (0, 0)),
            pl.BlockSpec((2 * H, H), lambda i: (0, 0)),
            pl.BlockSpec((2 * H, H), lambda i: (0, 0)),
            pl.BlockSpec((2 * H, 1), lambda i: (0, 0)),
            pl.BlockSpec((1, 1), lambda i: (0, 0)),
        ],
        out_specs=pl.BlockSpec((B, 1), lambda i: (0, 0)),
        out_shape=jax.ShapeDtypeStruct((B, 1), jnp.float32),
        scratch_shapes=[
            pltpu.VMEM((B, H), jnp.float32),
            pltpu.VMEM((B, D), jnp.float32),
        ],
    )(f1raw, m2sum, x0row, prep_W, prep_b.reshape(1, D), a1_Wx, a1_Wn,
      a2_Wx, a2_Wn, fc_W, fc_b.reshape(1, 1))


def kernel(ids, adj, emb, prep_W, prep_b, a1_Wx, a1_Wn, a2_Wx, a2_Wn, fc_W, fc_b):
    ids = ids.astype(jnp.int32)
    adj = adj.astype(jnp.int32)
    f1raw, m2sum = _sc_gather(ids, adj, emb)
    x0row = lax.slice(emb, (emb.shape[0] - 1, 0), (emb.shape[0], emb.shape[1]))
    return _tc_dense(f1raw, m2sum, x0row, prep_W, prep_b, a1_Wx, a1_Wn,
                     a2_Wx, a2_Wn, fc_W, fc_b, adj.shape[1])
